# baseline (speedup 1.0000x reference)
import jax
import jax.numpy as jnp
from jax.experimental import pallas as pl
from jax.experimental.pallas import tpu as pltpu

N = 1024
DH = 256
DE = 16
BI = 128
NB = N // BI

_BF = jnp.bfloat16


def _dot(p, q):
    return jnp.dot(p.astype(_BF), q.astype(_BF),
                   preferred_element_type=jnp.float32)


def _dot32(p, q):
    return jnp.dot(p, q, preferred_element_type=jnp.float32)


def _sig(a):
    return 0.5 + 0.5 * jnp.tanh(0.5 * a)


def _fused_body(adj_ref, et_ref, et2_ref, we_ref, h_ref, wmsg_ref, wz_ref, uz_ref,
                wr_ref, ur_ref, wh_ref, uh_ref, bz_ref, br_ref, bh_ref,
                wg_ref, bg_ref, wo_ref, bo_ref, node_ref, wnemb_ref, wfc_ref,
                bfc_ref, out_ref, emsg_ref, xw_ref, x1_ref):
    i = pl.program_id(0)
    inv_n = 1.0 / N
    rows = pl.ds(i * BI, BI)

    @pl.when(i == 0)
    def _():
        xw_ref[...] = _dot(h_ref[...], wmsg_ref[...])

    a = adj_ref[rows, :]
    ab = a[:, None, :]
    eagg = jnp.concatenate(
        [jnp.sum(et_ref[...] * ab, axis=2),
         jnp.sum(et2_ref[...] * ab, axis=2)], axis=1)
    em = _dot32(eagg, we_ref[...]) * inv_n
    emsg_ref[rows, :] = em

    hb = h_ref[rows, :]
    m1 = _dot(a, xw_ref[...]) * inv_n + em
    z1 = _sig(_dot(m1, wz_ref[...]) + _dot(hb, uz_ref[...])
                        + bz_ref[...])
    r1 = _sig(_dot(m1, wr_ref[...]) + _dot(hb, ur_ref[...])
                        + br_ref[...])
    hh1 = jnp.tanh(_dot(m1, wh_ref[...]) + _dot(r1 * hb, uh_ref[...])
                   + bh_ref[...])
    x1_ref[rows, :] = (1.0 - z1) * hb + z1 * hh1

    @pl.when(i == NB - 1)
    def _():
        adjm = adj_ref[...].astype(_BF)
        emsg = emsg_ref[...]
        x = x1_ref[...]
        for _ in range(2):
            m = _dot(adjm, _dot(x, wmsg_ref[...])) * inv_n + emsg
            z = _sig(_dot(m, wz_ref[...]) + _dot(x, uz_ref[...])
                               + bz_ref[...])
            r = _sig(_dot(m, wr_ref[...]) + _dot(x, ur_ref[...])
                               + br_ref[...])
            hh = jnp.tanh(_dot(m, wh_ref[...]) + _dot(r * x, uh_ref[...])
                          + bh_ref[...])
            x = (1.0 - z) * x + z * hh
        gate = _sig(_dot(x, wg_ref[...]) + bg_ref[...])
        hv = gate * jnp.tanh(_dot(x, wo_ref[...]) + bo_ref[...])
        gv = jnp.sum(hv, axis=0, keepdims=True)
        ne = _dot32(node_ref[...], wnemb_ref[...])
        out_ref[...] = (_dot32(gv, wfc_ref[:DH, :]) + _dot32(ne, wfc_ref[DH:, :])
                        + bfc_ref[...])


def kernel(h, e, adj, node, W_msg, W_e, Wz, Uz, Wr, Ur, Wh, Uh, bz, br, bh,
           W_g, b_g, W_o, b_o, W_nemb, W_fc, b_fc):
    adj2 = adj.reshape(N, N)
    et = jnp.transpose(e.reshape(N, N, DE), (0, 2, 1))
    h2 = h.reshape(N, DH)

    full = lambda *shape: pl.BlockSpec(shape, lambda i: (0,) * len(shape))
    out = pl.pallas_call(
        _fused_body,
        grid=(NB,),
        in_specs=[
            full(N, N),
            pl.BlockSpec((BI, DE // 2, N), lambda i: (i, 0, 0)),
            pl.BlockSpec((BI, DE // 2, N), lambda i: (i, 1, 0)),
            full(DE, DH),
            full(N, DH),
            full(DH, DH), full(DH, DH), full(DH, DH),
            full(DH, DH), full(DH, DH), full(DH, DH),
            full(DH, DH),
            full(1, DH), full(1, DH), full(1, DH),
            full(DH, DH), full(1, DH),
            full(DH, DH), full(1, DH),
            full(1, 128), full(128, DH),
            full(2 * DH, DH), full(1, DH),
        ],
        out_specs=full(1, DH),
        out_shape=jax.ShapeDtypeStruct((1, DH), jnp.float32),
        scratch_shapes=[pltpu.VMEM((N, DH), jnp.float32),
                        pltpu.VMEM((N, DH), jnp.float32),
                        pltpu.VMEM((N, DH), jnp.float32)],
    )(adj2, et, et, W_e, h2, W_msg, Wz, Uz, Wr, Ur, Wh, Uh,
      bz.reshape(1, DH), br.reshape(1, DH), bh.reshape(1, DH),
      W_g, b_g.reshape(1, DH), W_o, b_o.reshape(1, DH),
      node.reshape(1, 128), W_nemb, W_fc, b_fc.reshape(1, DH))

    return out.reshape(DH)


# speedup vs baseline: 1.0363x; 1.0363x over previous
import jax
import jax.numpy as jnp
from jax.experimental import pallas as pl
from jax.experimental.pallas import tpu as pltpu

N = 1024
DH = 256
DE = 16
BI = 128
NB = N // BI

_BF = jnp.bfloat16


def _dot(p, q):
    return jnp.dot(p.astype(_BF), q.astype(_BF),
                   preferred_element_type=jnp.float32)


def _dot32(p, q):
    return jnp.dot(p, q, preferred_element_type=jnp.float32)


def _sig(a):
    return 0.5 + 0.5 * jnp.tanh(0.5 * a)


def _fused_body(adj_ref, et_ref, we_ref, h_ref, wmsg_ref, wz_ref, uz_ref,
                wr_ref, ur_ref, wh_ref, uh_ref, bz_ref, br_ref, bh_ref,
                wg_ref, bg_ref, wo_ref, bo_ref, node_ref, wnemb_ref, wfc_ref,
                bfc_ref, out_ref, emsg_ref, xw_ref, x1_ref):
    i = pl.program_id(0)
    inv_n = 1.0 / N
    rows = pl.ds(i * BI, BI)

    @pl.when(i == 0)
    def _():
        xw_ref[...] = _dot(h_ref[...], wmsg_ref[...])

    a = adj_ref[rows, :]
    eagg = jnp.sum(et_ref[...] * a[:, None, :], axis=2)
    em = _dot32(eagg, we_ref[...]) * inv_n
    emsg_ref[rows, :] = em

    hb = h_ref[rows, :]
    m1 = _dot(a, xw_ref[...]) * inv_n + em
    z1 = _sig(_dot(m1, wz_ref[...]) + _dot(hb, uz_ref[...])
                        + bz_ref[...])
    r1 = _sig(_dot(m1, wr_ref[...]) + _dot(hb, ur_ref[...])
                        + br_ref[...])
    hh1 = jnp.tanh(_dot(m1, wh_ref[...]) + _dot(r1 * hb, uh_ref[...])
                   + bh_ref[...])
    x1_ref[rows, :] = (1.0 - z1) * hb + z1 * hh1

    @pl.when(i == NB - 1)
    def _():
        adjm = adj_ref[...].astype(_BF)
        emsg = emsg_ref[...]
        x = x1_ref[...]
        for _ in range(2):
            m = _dot(adjm, _dot(x, wmsg_ref[...])) * inv_n + emsg
            z = _sig(_dot(m, wz_ref[...]) + _dot(x, uz_ref[...])
                               + bz_ref[...])
            r = _sig(_dot(m, wr_ref[...]) + _dot(x, ur_ref[...])
                               + br_ref[...])
            hh = jnp.tanh(_dot(m, wh_ref[...]) + _dot(r * x, uh_ref[...])
                          + bh_ref[...])
            x = (1.0 - z) * x + z * hh
        gate = _sig(_dot(x, wg_ref[...]) + bg_ref[...])
        hv = gate * jnp.tanh(_dot(x, wo_ref[...]) + bo_ref[...])
        gv = jnp.sum(hv, axis=0, keepdims=True)
        ne = _dot32(node_ref[...], wnemb_ref[...])
        out_ref[...] = (_dot32(gv, wfc_ref[:DH, :]) + _dot32(ne, wfc_ref[DH:, :])
                        + bfc_ref[...])


def kernel(h, e, adj, node, W_msg, W_e, Wz, Uz, Wr, Ur, Wh, Uh, bz, br, bh,
           W_g, b_g, W_o, b_o, W_nemb, W_fc, b_fc):
    adj2 = adj.reshape(N, N)
    et = jnp.transpose(e.reshape(N, N, DE), (0, 2, 1))
    h2 = h.reshape(N, DH)

    full = lambda *shape: pl.BlockSpec(shape, lambda i: (0,) * len(shape))
    out = pl.pallas_call(
        _fused_body,
        grid=(NB,),
        in_specs=[
            full(N, N),
            pl.BlockSpec((BI, DE, N), lambda i: (i, 0, 0)),
            full(DE, DH),
            full(N, DH),
            full(DH, DH), full(DH, DH), full(DH, DH),
            full(DH, DH), full(DH, DH), full(DH, DH),
            full(DH, DH),
            full(1, DH), full(1, DH), full(1, DH),
            full(DH, DH), full(1, DH),
            full(DH, DH), full(1, DH),
            full(1, 128), full(128, DH),
            full(2 * DH, DH), full(1, DH),
        ],
        out_specs=full(1, DH),
        out_shape=jax.ShapeDtypeStruct((1, DH), jnp.float32),
        scratch_shapes=[pltpu.VMEM((N, DH), jnp.float32),
                        pltpu.VMEM((N, DH), jnp.float32),
                        pltpu.VMEM((N, DH), jnp.float32)],
    )(adj2, et, W_e, h2, W_msg, Wz, Uz, Wr, Ur, Wh, Uh,
      bz.reshape(1, DH), br.reshape(1, DH), bh.reshape(1, DH),
      W_g, b_g.reshape(1, DH), W_o, b_o.reshape(1, DH),
      node.reshape(1, 128), W_nemb, W_fc, b_fc.reshape(1, DH))

    return out.reshape(DH)
